# baseline (speedup 1.0000x reference)
import functools

import jax
import jax.numpy as jnp
from jax import lax
from jax.experimental import pallas as pl
from jax.experimental.pallas import tpu as pltpu
from jax.experimental.pallas import tpu_sc as plsc

BATCH = 16384
HIST = 50
VOCAB = 1000000
DIM = 32
NC = 2
NS = 16
NW = NC * NS
BAGS_PER_W = BATCH // NW
CHUNK = 32
NCHUNK = BAGS_PER_W // CHUNK
SCALE = 1.0 / HIST

VB = 128
NFULL = VOCAB // VB
TAIL = VOCAB - NFULL * VB
TAIL_W = 4


NBLK = 244
NPAIR = NBLK // 2
NEXTRA = NFULL - NBLK * NW


def _transpose_block(in_ref, blk_ref, iota, perms, sidxb, nlanes):
    rows = (iota, iota + 16)

    def sub(v32, carry):
        for half in range(2):
            v0 = v32 * 32 + half * 16
            for jb in range(2):
                rowv = rows[jb]
                for h in range(0, 16, 4):
                    lanevs = [perms[h + s] + v0 for s in range(4)]
                    vals = [plsc.load_gather(in_ref, [rowv, lanevs[s]])
                            for s in range(4)]
                    for s in range(4):
                        plsc.store_scatter(
                            blk_ref, [lanevs[s] * DIM + rowv], vals[s]
                        )
        return carry

    lax.fori_loop(0, nlanes // 32, sub, 0)


def _detile_body(wt_hbm, out_hbm, in0_v, in1_v, b0_v, b1_v, tin_v, tblk_v,
                 si0, si1, so0, so1):
    wid = lax.axis_index("s") * NC + lax.axis_index("c")
    iota = jnp.arange(16, dtype=jnp.int32)
    perms = [(iota + s) & 15 for s in range(16)]
    sidxb = [p * DIM + iota for p in perms]
    BLK = VB * DIM

    def in_dma(k, buf, sem):
        return pltpu.make_async_copy(wt_hbm.at[:, pl.ds(k * VB, VB)], buf, sem)

    def out_dma(k, buf, sem):
        return pltpu.make_async_copy(buf, out_hbm.at[pl.ds(k * BLK, BLK)], sem)

    def pair_body(p, carry):
        k0 = wid + (2 * p) * NW
        k1 = k0 + NW
        in_dma(k0, in0_v, si0).start()
        in_dma(k1, in1_v, si1).start()
        in_dma(k0, in0_v, si0).wait()

        @pl.when(p > 0)
        def _w0():
            out_dma(k0 - 2 * NW, b0_v, so0).wait()

        _transpose_block(in0_v, b0_v, iota, perms, sidxb, VB)
        out_dma(k0, b0_v, so0).start()
        in_dma(k1, in1_v, si1).wait()

        @pl.when(p > 0)
        def _w1():
            out_dma(k1 - 2 * NW, b1_v, so1).wait()

        _transpose_block(in1_v, b1_v, iota, perms, sidxb, VB)
        out_dma(k1, b1_v, so1).start()
        return carry

    lax.fori_loop(0, NPAIR, pair_body, 0)
    klast = wid + (NBLK - 2) * NW
    out_dma(klast, b0_v, so0).wait()
    out_dma(klast + NW, b1_v, so1).wait()

    @pl.when(wid < NEXTRA)
    def _extra():
        k = wid + NBLK * NW
        pltpu.sync_copy(wt_hbm.at[:, pl.ds(k * VB, VB)], in0_v)
        _transpose_block(in0_v, b0_v, iota, perms, sidxb, VB)
        pltpu.sync_copy(b0_v, out_hbm.at[pl.ds(k * BLK, BLK)])

    @pl.when(wid == TAIL_W)
    def _tail():
        pltpu.sync_copy(wt_hbm.at[:, pl.ds(NFULL * VB, TAIL)], tin_v)
        _transpose_block(tin_v, tblk_v, iota, perms, sidxb, TAIL)
        pltpu.sync_copy(
            tblk_v, out_hbm.at[pl.ds(NFULL * VB * DIM, TAIL * DIM)]
        )


def _emb_bag_body(idx_hbm, table_hbm, out_hbm,
                  idx0_v, idx1_v, rows0_v, rows1_v, out0_v, out1_v,
                  sem0, sem1):
    wid = lax.axis_index("s") * NC + lax.axis_index("c")
    bag_base = wid * BAGS_PER_W

    def fire(bag0, idx_v, rows_v, sem):
        pltpu.sync_copy(idx_hbm.at[pl.ds(bag0, CHUNK)], idx_v)
        for i in range(CHUNK):
            pltpu.async_copy(table_hbm.at[idx_v.at[i]], rows_v.at[i], sem)

    def drain(idx_v, rows_v, sem):
        for i in range(CHUNK):
            pltpu.make_async_copy(
                table_hbm.at[idx_v.at[i]], rows_v.at[i], sem
            ).wait()

    def reduce(bag0, rows_v, out_v):
        def bag_body(i, carry2):
            acc0 = rows_v[i, 0, pl.ds(0, 16)]
            acc1 = rows_v[i, 0, pl.ds(16, 16)]
            for j in range(1, HIST):
                acc0 = acc0 + rows_v[i, j, pl.ds(0, 16)]
                acc1 = acc1 + rows_v[i, j, pl.ds(16, 16)]
            out_v[i, pl.ds(0, 16)] = acc0 * SCALE
            out_v[i, pl.ds(16, 16)] = acc1 * SCALE
            return carry2

        lax.fori_loop(0, CHUNK, bag_body, 0)
        pltpu.sync_copy(out_v, out_hbm.at[pl.ds(bag0, CHUNK)])

    fire(bag_base, idx0_v, rows0_v, sem0)

    def pair_body(p, carry):
        c0 = bag_base + (2 * p) * CHUNK
        c1 = c0 + CHUNK
        fire(c1, idx1_v, rows1_v, sem1)
        drain(idx0_v, rows0_v, sem0)
        reduce(c0, rows0_v, out0_v)

        @pl.when(p < NCHUNK // 2 - 1)
        def _next():
            fire(c1 + CHUNK, idx0_v, rows0_v, sem0)

        drain(idx1_v, rows1_v, sem1)
        reduce(c1, rows1_v, out1_v)
        return carry

    lax.fori_loop(0, NCHUNK // 2, pair_body, 0)


def kernel(input, weight):
    idx = input.astype(jnp.int32)
    mesh = plsc.VectorSubcoreMesh(core_axis_name="c", subcore_axis_name="s")

    detile = functools.partial(
        pl.kernel,
        mesh=mesh,
        compiler_params=pltpu.CompilerParams(
            use_tc_tiling_on_sc=True, needs_layout_passes=False
        ),
        out_type=jax.ShapeDtypeStruct((VOCAB * DIM,), jnp.float32),
        scratch_types=[
            pltpu.VMEM((DIM, VB), jnp.float32),
            pltpu.VMEM((DIM, VB), jnp.float32),
            pltpu.VMEM((VB * DIM,), jnp.float32),
            pltpu.VMEM((VB * DIM,), jnp.float32),
            pltpu.VMEM((DIM, TAIL), jnp.float32),
            pltpu.VMEM((TAIL * DIM,), jnp.float32),
            pltpu.SemaphoreType.DMA,
            pltpu.SemaphoreType.DMA,
            pltpu.SemaphoreType.DMA,
            pltpu.SemaphoreType.DMA,
        ],
    )(_detile_body)
    table = detile(weight.T).reshape(VOCAB, DIM)

    emb = functools.partial(
        pl.kernel,
        mesh=mesh,
        compiler_params=pltpu.CompilerParams(use_tc_tiling_on_sc=False),
        out_type=jax.ShapeDtypeStruct((BATCH, DIM), jnp.float32),
        scratch_types=[
            pltpu.VMEM((CHUNK, HIST), jnp.int32),
            pltpu.VMEM((CHUNK, HIST), jnp.int32),
            pltpu.VMEM((CHUNK, HIST, DIM), jnp.float32),
            pltpu.VMEM((CHUNK, HIST, DIM), jnp.float32),
            pltpu.VMEM((CHUNK, DIM), jnp.float32),
            pltpu.VMEM((CHUNK, DIM), jnp.float32),
            pltpu.SemaphoreType.DMA,
            pltpu.SemaphoreType.DMA,
        ],
    )(_emb_bag_body)
    return emb(idx, table)


# speedup vs baseline: 1.0570x; 1.0570x over previous
import functools

import jax
import jax.numpy as jnp
from jax import lax
from jax.experimental import pallas as pl
from jax.experimental.pallas import tpu as pltpu
from jax.experimental.pallas import tpu_sc as plsc

BATCH = 16384
HIST = 50
VOCAB = 1000000
DIM = 32
NC = 2
NS = 16
NW = NC * NS
BAGS_PER_W = BATCH // NW
CHUNK = 32
NCHUNK = BAGS_PER_W // CHUNK
SCALE = 1.0 / HIST

VB = 256
NFULL = VOCAB // VB
TAIL = VOCAB - NFULL * VB
TAIL_W = 4


NBLK = 122
NPAIR = NBLK // 2
NEXTRA = NFULL - NBLK * NW


def _transpose_block(in_ref, blk_ref, iota, perms, sidxb, nlanes):
    rows = (iota, iota + 16)

    def sub(v16, carry):
        v0 = v16 * 16
        base = iota + (v16 >> 31)
        for jb in range(2):
            rowv = rows[jb]
            for h in range(0, 16, 8):
                lanevs = [((base + (h + s)) & 15) + v0 for s in range(8)]
                vals = [plsc.load_gather(in_ref, [rowv, lanevs[s]])
                        for s in range(8)]
                for s in range(8):
                    plsc.store_scatter(
                        blk_ref, [lanevs[s] * DIM + rowv], vals[s]
                    )
        return carry

    lax.fori_loop(0, nlanes // 16, sub, 0)


def _detile_body(wt_hbm, out_hbm, in0_v, in1_v, b0_v, b1_v, tin_v, tblk_v,
                 si0, si1, so0, so1):
    wid = lax.axis_index("s") * NC + lax.axis_index("c")
    iota = jnp.arange(16, dtype=jnp.int32)
    perms = [(iota + s) & 15 for s in range(16)]
    sidxb = [p * DIM + iota for p in perms]
    BLK = VB * DIM

    def in_dma(k, buf, sem):
        return pltpu.make_async_copy(wt_hbm.at[:, pl.ds(k * VB, VB)], buf, sem)

    def out_dma(k, buf, sem):
        return pltpu.make_async_copy(buf, out_hbm.at[pl.ds(k * BLK, BLK)], sem)

    def pair_body(p, carry):
        k0 = wid + (2 * p) * NW
        k1 = k0 + NW
        in_dma(k0, in0_v, si0).start()
        in_dma(k1, in1_v, si1).start()
        in_dma(k0, in0_v, si0).wait()

        @pl.when(p > 0)
        def _w0():
            out_dma(k0 - 2 * NW, b0_v, so0).wait()

        _transpose_block(in0_v, b0_v, iota, perms, sidxb, VB)
        out_dma(k0, b0_v, so0).start()
        in_dma(k1, in1_v, si1).wait()

        @pl.when(p > 0)
        def _w1():
            out_dma(k1 - 2 * NW, b1_v, so1).wait()

        _transpose_block(in1_v, b1_v, iota, perms, sidxb, VB)
        out_dma(k1, b1_v, so1).start()
        return carry

    lax.fori_loop(0, NPAIR, pair_body, 0)
    klast = wid + (NBLK - 2) * NW
    out_dma(klast, b0_v, so0).wait()
    out_dma(klast + NW, b1_v, so1).wait()

    @pl.when(wid < NEXTRA)
    def _extra():
        k = wid + NBLK * NW
        pltpu.sync_copy(wt_hbm.at[:, pl.ds(k * VB, VB)], in0_v)
        _transpose_block(in0_v, b0_v, iota, perms, sidxb, VB)
        pltpu.sync_copy(b0_v, out_hbm.at[pl.ds(k * BLK, BLK)])

    @pl.when(wid == TAIL_W)
    def _tail():
        pltpu.sync_copy(wt_hbm.at[:, pl.ds(NFULL * VB, TAIL)], tin_v)
        _transpose_block(tin_v, tblk_v, iota, perms, sidxb, TAIL)
        pltpu.sync_copy(
            tblk_v, out_hbm.at[pl.ds(NFULL * VB * DIM, TAIL * DIM)]
        )


def _emb_bag_body(idx_hbm, table_hbm, out_hbm,
                  idx0_v, idx1_v, rows0_v, rows1_v, out0_v, out1_v,
                  sem0, sem1):
    wid = lax.axis_index("s") * NC + lax.axis_index("c")
    bag_base = wid * BAGS_PER_W

    def fire(bag0, idx_v, rows_v, sem):
        pltpu.sync_copy(idx_hbm.at[pl.ds(bag0, CHUNK)], idx_v)
        for i in range(CHUNK):
            pltpu.async_copy(table_hbm.at[idx_v.at[i]], rows_v.at[i], sem)

    def drain(idx_v, rows_v, sem):
        for i in range(CHUNK):
            pltpu.make_async_copy(
                table_hbm.at[idx_v.at[i]], rows_v.at[i], sem
            ).wait()

    def reduce(bag0, rows_v, out_v):
        def bag_body(i, carry2):
            acc0 = rows_v[i, 0, pl.ds(0, 16)]
            acc1 = rows_v[i, 0, pl.ds(16, 16)]
            for j in range(1, HIST):
                acc0 = acc0 + rows_v[i, j, pl.ds(0, 16)]
                acc1 = acc1 + rows_v[i, j, pl.ds(16, 16)]
            out_v[i, pl.ds(0, 16)] = acc0 * SCALE
            out_v[i, pl.ds(16, 16)] = acc1 * SCALE
            return carry2

        lax.fori_loop(0, CHUNK, bag_body, 0)
        pltpu.sync_copy(out_v, out_hbm.at[pl.ds(bag0, CHUNK)])

    fire(bag_base, idx0_v, rows0_v, sem0)

    def pair_body(p, carry):
        c0 = bag_base + (2 * p) * CHUNK
        c1 = c0 + CHUNK
        fire(c1, idx1_v, rows1_v, sem1)
        drain(idx0_v, rows0_v, sem0)
        reduce(c0, rows0_v, out0_v)

        @pl.when(p < NCHUNK // 2 - 1)
        def _next():
            fire(c1 + CHUNK, idx0_v, rows0_v, sem0)

        drain(idx1_v, rows1_v, sem1)
        reduce(c1, rows1_v, out1_v)
        return carry

    lax.fori_loop(0, NCHUNK // 2, pair_body, 0)


def kernel(input, weight):
    idx = input.astype(jnp.int32)
    mesh = plsc.VectorSubcoreMesh(core_axis_name="c", subcore_axis_name="s")

    detile = functools.partial(
        pl.kernel,
        mesh=mesh,
        compiler_params=pltpu.CompilerParams(
            use_tc_tiling_on_sc=True, needs_layout_passes=False
        ),
        out_type=jax.ShapeDtypeStruct((VOCAB * DIM,), jnp.float32),
        scratch_types=[
            pltpu.VMEM((DIM, VB), jnp.float32),
            pltpu.VMEM((DIM, VB), jnp.float32),
            pltpu.VMEM((VB * DIM,), jnp.float32),
            pltpu.VMEM((VB * DIM,), jnp.float32),
            pltpu.VMEM((DIM, TAIL), jnp.float32),
            pltpu.VMEM((TAIL * DIM,), jnp.float32),
            pltpu.SemaphoreType.DMA,
            pltpu.SemaphoreType.DMA,
            pltpu.SemaphoreType.DMA,
            pltpu.SemaphoreType.DMA,
        ],
    )(_detile_body)
    table = detile(weight.T).reshape(VOCAB, DIM)

    emb = functools.partial(
        pl.kernel,
        mesh=mesh,
        compiler_params=pltpu.CompilerParams(use_tc_tiling_on_sc=False),
        out_type=jax.ShapeDtypeStruct((BATCH, DIM), jnp.float32),
        scratch_types=[
            pltpu.VMEM((CHUNK, HIST), jnp.int32),
            pltpu.VMEM((CHUNK, HIST), jnp.int32),
            pltpu.VMEM((CHUNK, HIST, DIM), jnp.float32),
            pltpu.VMEM((CHUNK, HIST, DIM), jnp.float32),
            pltpu.VMEM((CHUNK, DIM), jnp.float32),
            pltpu.VMEM((CHUNK, DIM), jnp.float32),
            pltpu.SemaphoreType.DMA,
            pltpu.SemaphoreType.DMA,
        ],
    )(_emb_bag_body)
    return emb(idx, table)


# speedup vs baseline: 1.0621x; 1.0048x over previous
import functools

import jax
import jax.numpy as jnp
from jax import lax
from jax.experimental import pallas as pl
from jax.experimental.pallas import tpu as pltpu
from jax.experimental.pallas import tpu_sc as plsc

BATCH = 16384
HIST = 50
VOCAB = 1000000
DIM = 32
NC = 2
NS = 16
NW = NC * NS
BAGS_PER_W = BATCH // NW
CHUNK = 32
NCHUNK = BAGS_PER_W // CHUNK
SCALE = 1.0 / HIST

VB = 256
NFULL = VOCAB // VB
TAIL = VOCAB - NFULL * VB
TAIL_W = 4


NBLK = 122
NPAIR = NBLK // 2
NEXTRA = NFULL - NBLK * NW


def _transpose_block(in_ref, blk_ref, iota, nlanes):
    rows = (iota, iota + 16)

    def sub(v16, carry):
        v0 = v16 * 16
        base = iota + (v16 >> 31)
        for jb in range(2):
            rowv = rows[jb]
            for h in range(0, 16, 8):
                lanevs = [((base + (h + s)) & 15) + v0 for s in range(8)]
                vals = [plsc.load_gather(in_ref, [rowv, lanevs[s]])
                        for s in range(8)]
                for s in range(8):
                    plsc.store_scatter(
                        blk_ref, [lanevs[s] * DIM + rowv], vals[s]
                    )
        return carry

    lax.fori_loop(0, nlanes // 16, sub, 0)


def _detile_body(wt_hbm, out_hbm, in0_v, in1_v, b0_v, b1_v, tin_v, tblk_v,
                 si0, si1, so0, so1):
    wid = lax.axis_index("s") * NC + lax.axis_index("c")
    iota = jnp.arange(16, dtype=jnp.int32)
    BLK = VB * DIM

    def in_dma(k, buf, sem):
        return pltpu.make_async_copy(wt_hbm.at[:, pl.ds(k * VB, VB)], buf, sem)

    def out_dma(k, buf, sem):
        return pltpu.make_async_copy(buf, out_hbm.at[pl.ds(k * BLK, BLK)], sem)

    def pair_body(p, carry):
        k0 = wid + (2 * p) * NW
        k1 = k0 + NW
        in_dma(k0, in0_v, si0).start()
        in_dma(k1, in1_v, si1).start()
        in_dma(k0, in0_v, si0).wait()

        @pl.when(p > 0)
        def _w0():
            out_dma(k0 - 2 * NW, b0_v, so0).wait()

        _transpose_block(in0_v, b0_v, iota, VB)
        out_dma(k0, b0_v, so0).start()
        in_dma(k1, in1_v, si1).wait()

        @pl.when(p > 0)
        def _w1():
            out_dma(k1 - 2 * NW, b1_v, so1).wait()

        _transpose_block(in1_v, b1_v, iota, VB)
        out_dma(k1, b1_v, so1).start()
        return carry

    lax.fori_loop(0, NPAIR, pair_body, 0)
    klast = wid + (NBLK - 2) * NW
    out_dma(klast, b0_v, so0).wait()
    out_dma(klast + NW, b1_v, so1).wait()

    @pl.when(wid < NEXTRA)
    def _extra():
        k = wid + NBLK * NW
        pltpu.sync_copy(wt_hbm.at[:, pl.ds(k * VB, VB)], in0_v)
        _transpose_block(in0_v, b0_v, iota, VB)
        pltpu.sync_copy(b0_v, out_hbm.at[pl.ds(k * BLK, BLK)])

    @pl.when(wid == TAIL_W)
    def _tail():
        pltpu.sync_copy(wt_hbm.at[:, pl.ds(NFULL * VB, TAIL)], tin_v)
        _transpose_block(tin_v, tblk_v, iota, TAIL)
        pltpu.sync_copy(
            tblk_v, out_hbm.at[pl.ds(NFULL * VB * DIM, TAIL * DIM)]
        )


def _emb_bag_body(idx_hbm, table_hbm, out_hbm,
                  idx0_v, idx1_v, rows0_v, rows1_v, out0_v, out1_v,
                  sem0, sem1):
    wid = lax.axis_index("s") * NC + lax.axis_index("c")
    bag_base = wid * BAGS_PER_W

    def fire(bag0, idx_v, rows_v, sem):
        pltpu.sync_copy(idx_hbm.at[pl.ds(bag0, CHUNK)], idx_v)
        for i in range(CHUNK):
            pltpu.async_copy(table_hbm.at[idx_v.at[i]], rows_v.at[i], sem)

    def drain(idx_v, rows_v, sem):
        for i in range(CHUNK):
            pltpu.make_async_copy(
                table_hbm.at[idx_v.at[i]], rows_v.at[i], sem
            ).wait()

    def reduce(bag0, rows_v, out_v):
        def bag_body(i, carry2):
            acc0 = rows_v[i, 0, pl.ds(0, 16)]
            acc1 = rows_v[i, 0, pl.ds(16, 16)]
            for j in range(1, HIST):
                acc0 = acc0 + rows_v[i, j, pl.ds(0, 16)]
                acc1 = acc1 + rows_v[i, j, pl.ds(16, 16)]
            out_v[i, pl.ds(0, 16)] = acc0 * SCALE
            out_v[i, pl.ds(16, 16)] = acc1 * SCALE
            return carry2

        lax.fori_loop(0, CHUNK, bag_body, 0)
        pltpu.sync_copy(out_v, out_hbm.at[pl.ds(bag0, CHUNK)])

    fire(bag_base, idx0_v, rows0_v, sem0)

    def pair_body(p, carry):
        c0 = bag_base + (2 * p) * CHUNK
        c1 = c0 + CHUNK
        fire(c1, idx1_v, rows1_v, sem1)
        drain(idx0_v, rows0_v, sem0)
        reduce(c0, rows0_v, out0_v)

        @pl.when(p < NCHUNK // 2 - 1)
        def _next():
            fire(c1 + CHUNK, idx0_v, rows0_v, sem0)

        drain(idx1_v, rows1_v, sem1)
        reduce(c1, rows1_v, out1_v)
        return carry

    lax.fori_loop(0, NCHUNK // 2, pair_body, 0)


def kernel(input, weight):
    idx = input.astype(jnp.int32)
    mesh = plsc.VectorSubcoreMesh(core_axis_name="c", subcore_axis_name="s")

    detile = functools.partial(
        pl.kernel,
        mesh=mesh,
        compiler_params=pltpu.CompilerParams(
            use_tc_tiling_on_sc=True, needs_layout_passes=False
        ),
        out_type=jax.ShapeDtypeStruct((VOCAB * DIM,), jnp.float32),
        scratch_types=[
            pltpu.VMEM((DIM, VB), jnp.float32),
            pltpu.VMEM((DIM, VB), jnp.float32),
            pltpu.VMEM((VB * DIM,), jnp.float32),
            pltpu.VMEM((VB * DIM,), jnp.float32),
            pltpu.VMEM((DIM, TAIL), jnp.float32),
            pltpu.VMEM((TAIL * DIM,), jnp.float32),
            pltpu.SemaphoreType.DMA,
            pltpu.SemaphoreType.DMA,
            pltpu.SemaphoreType.DMA,
            pltpu.SemaphoreType.DMA,
        ],
    )(_detile_body)
    table = detile(weight.T).reshape(VOCAB, DIM)

    emb = functools.partial(
        pl.kernel,
        mesh=mesh,
        compiler_params=pltpu.CompilerParams(use_tc_tiling_on_sc=False),
        out_type=jax.ShapeDtypeStruct((BATCH, DIM), jnp.float32),
        scratch_types=[
            pltpu.VMEM((CHUNK, HIST), jnp.int32),
            pltpu.VMEM((CHUNK, HIST), jnp.int32),
            pltpu.VMEM((CHUNK, HIST, DIM), jnp.float32),
            pltpu.VMEM((CHUNK, HIST, DIM), jnp.float32),
            pltpu.VMEM((CHUNK, DIM), jnp.float32),
            pltpu.VMEM((CHUNK, DIM), jnp.float32),
            pltpu.SemaphoreType.DMA,
            pltpu.SemaphoreType.DMA,
        ],
    )(_emb_bag_body)
    return emb(idx, table)
